# baseline (speedup 1.0000x reference)
import functools

import jax
import jax.numpy as jnp
from jax import lax
from jax.experimental import pallas as pl
from jax.experimental.pallas import tpu as pltpu
from jax.experimental.pallas import tpu_sc as plsc

_N = 10000
_E = 320000
_D = 128

_DW = 144
_B = 40
_NW = 32
_EPW = _E // _NW
_NBLK = _EPW // _B
_NPAD = 10240
_RPS = _NPAD // 16


def _proj_body(x_ref, ws_ref, wd_ref, ps_ref, pd_ref):
    x = x_ref[...]
    ps_ref[...] = jnp.dot(x, ws_ref[...], preferred_element_type=jnp.float32)
    pd_ref[...] = jnp.dot(x, wd_ref[...], preferred_element_type=jnp.float32)


def _project(meta_xs, w_src, w_dst):
    blk = 1000
    wspec = pl.BlockSpec((_D, 2 * _D), lambda i: (0, 0))
    ospec = pl.BlockSpec((blk, 2 * _D), lambda i: (i, 0))
    return pl.pallas_call(
        _proj_body,
        grid=(_N // blk,),
        in_specs=[pl.BlockSpec((blk, _D), lambda i: (i, 0)), wspec, wspec],
        out_specs=[ospec, ospec],
        out_shape=[
            jax.ShapeDtypeStruct((_N, 2 * _D), jnp.float32),
            jax.ShapeDtypeStruct((_N, 2 * _D), jnp.float32),
        ],
    )(meta_xs, w_src, w_dst)


def _pack_pair(g, v):
    def rnd(x):
        u = lax.bitcast_convert_type(x, jnp.int32)
        return u + jnp.int32(0x7FFF) + jnp.bitwise_and(
            lax.shift_right_logical(u, 16), jnp.int32(1))
    gr = lax.shift_right_logical(rnd(g), 16)
    vr = jnp.bitwise_and(rnd(v), jnp.int32(-65536))
    return jnp.bitwise_or(vr, gr)


def _edge_body(ef_ref, wg_ref, wv_ref, bg_ref, bv_ref, out_ref):
    x = ef_ref[...]
    out_ref[...] = _pack_pair(
        jnp.dot(x, wg_ref[...], preferred_element_type=jnp.float32)
        + bg_ref[...],
        jnp.dot(x, wv_ref[...], preferred_element_type=jnp.float32)
        + bv_ref[...])


def _edge_mlp(edge_feature, wg, wv, bg, bv):
    blk = 2000
    wspec = pl.BlockSpec((_D, _D), lambda i: (0, 0))
    bspec = pl.BlockSpec((1, _D), lambda i: (0, 0))
    return pl.pallas_call(
        _edge_body,
        grid=(_E // blk,),
        in_specs=[pl.BlockSpec((blk, _D), lambda i: (i, 0)),
                  wspec, wspec, bspec, bspec],
        out_specs=pl.BlockSpec((blk, _D), lambda i: (i, 0)),
        out_shape=jax.ShapeDtypeStruct((_E, _D), jnp.int32),
    )(edge_feature, wg, wv, bg, bv)


_BB = 16
_NBLK2 = _EPW // _BB
_PAIRS = (_NBLK2 - 1) // 2


def _sc_msg_body(psrc, pdst, eb, srci, dsti, zrows, out_sum,
                 sidx, didx, eb0, eb1, ps0, ps1, pd0, pd1, mg0, mg1,
                 sc0, sc1, acc, sg0, sg1, ss0, ss1):
    c = lax.axis_index("c")
    s = lax.axis_index("s")
    wid = s * 2 + c

    ebufs = (eb0, eb1)
    psbs = (ps0, ps1)
    pdbs = (pd0, pd1)
    mgbs = (mg0, mg1)
    scidx = (sc0, sc1)
    sgs = (sg0, sg1)
    sss = (ss0, ss1)

    pltpu.sync_copy(zrows.at[pl.ds(s * _RPS, _RPS)],
                    acc.at[pl.ds(s * _RPS, _RPS)])
    plsc.subcore_barrier()

    base_e = wid * _EPW
    pltpu.sync_copy(srci.at[pl.ds(base_e, _EPW)], sidx)
    pltpu.sync_copy(dsti.at[pl.ds(base_e, _EPW)], didx)

    def issue(b, st):
        pltpu.async_copy(eb.at[pl.ds(base_e + b * _BB, _BB)], ebufs[st],
                         sgs[st])
        pltpu.async_copy(psrc.at[sidx.at[pl.ds(b * _BB, _BB)]], psbs[st],
                         sgs[st])
        pltpu.async_copy(pdst.at[didx.at[pl.ds(b * _BB, _BB)]], pdbs[st],
                         sgs[st])

    def wait_gathers(st):
        pltpu.make_async_copy(eb.at[pl.ds(0, _BB)], ebufs[st], sgs[st]).wait()
        pltpu.make_async_copy(psrc.at[pl.ds(0, _BB)], psbs[st],
                              sgs[st]).wait()
        pltpu.make_async_copy(psrc.at[pl.ds(0, _BB)], pdbs[st],
                              sgs[st]).wait()

    def drain_scatter(st):
        pltpu.make_async_copy(zrows.at[pl.ds(0, _BB)], mgbs[st],
                              sss[st]).wait()

    def compute(st):
        @plsc.parallel_loop(0, _BB, unroll=2)
        def do_edge(e):
            for j in range(8):
                o = j * 16
                xe = ebufs[st][e, pl.ds(o, 16)]
                g = (psbs[st][e, pl.ds(o, 16)] + pdbs[st][e, pl.ds(o, 16)]
                     + lax.bitcast_convert_type(lax.shift_left(xe, 16),
                                                jnp.float32))
                v = (psbs[st][e, pl.ds(o + _D, 16)]
                     + pdbs[st][e, pl.ds(o + _D, 16)]
                     + lax.bitcast_convert_type(
                         lax.bitwise_and(xe, jnp.int32(-65536)),
                         jnp.float32))
                mgbs[st][e, pl.ds(o, 16)] = v / (1.0 + jnp.exp(-g))

    def scatter(b, st):
        scidx[st][...] = sidx[pl.ds(b * _BB, _BB)]
        pltpu.async_copy(mgbs[st], acc.at[scidx[st]], sss[st], add=True)

    issue(0, 0)

    def pair(m, carry):
        b0 = 2 * m
        b1 = b0 + 1
        issue(b1, 1)
        wait_gathers(0)

        @pl.when(m >= 1)
        def _():
            drain_scatter(0)

        compute(0)
        scatter(b0, 0)
        issue(b0 + 2, 0)
        wait_gathers(1)

        @pl.when(m >= 1)
        def _():
            drain_scatter(1)

        compute(1)
        scatter(b1, 1)
        return carry

    lax.fori_loop(0, _PAIRS, pair, 0)
    wait_gathers(0)
    drain_scatter(0)
    compute(0)
    scatter(_NBLK2 - 1, 0)
    drain_scatter(0)
    drain_scatter(1)

    plsc.subcore_barrier()
    pltpu.sync_copy(acc.at[pl.ds(s * _RPS, _RPS)],
                    out_sum.at[pl.ds(c * _NPAD + s * _RPS, _RPS)])


def _sc_msg(psrc, pdst, eb, src, dst, zrows):
    mesh = plsc.VectorSubcoreMesh(core_axis_name="c", subcore_axis_name="s")
    fn = functools.partial(
        pl.kernel,
        mesh=mesh,
        out_type=jax.ShapeDtypeStruct((2 * _NPAD, _D), jnp.float32),
        scratch_types=[
            pltpu.VMEM((_EPW,), jnp.int32),
            pltpu.VMEM((_EPW,), jnp.int32),
            pltpu.VMEM((_BB, _D), jnp.int32),
            pltpu.VMEM((_BB, _D), jnp.int32),
            pltpu.VMEM((_BB, 2 * _D), jnp.float32),
            pltpu.VMEM((_BB, 2 * _D), jnp.float32),
            pltpu.VMEM((_BB, 2 * _D), jnp.float32),
            pltpu.VMEM((_BB, 2 * _D), jnp.float32),
            pltpu.VMEM((_BB, _D), jnp.float32),
            pltpu.VMEM((_BB, _D), jnp.float32),
            pltpu.VMEM((_BB,), jnp.int32),
            pltpu.VMEM((_BB,), jnp.int32),
            pltpu.VMEM_SHARED((_NPAD, _D), jnp.float32),
            pltpu.SemaphoreType.DMA,
            pltpu.SemaphoreType.DMA,
            pltpu.SemaphoreType.DMA,
            pltpu.SemaphoreType.DMA,
        ],
    )(_sc_msg_body)
    return fn(psrc, pdst, eb, src, dst, zrows)


def _sc_cnt_body(srci, zrows, out_cnt,
                 sidx, mg0, mg1, sc0, sc1, acc, ss0, ss1):
    c = lax.axis_index("c")
    s = lax.axis_index("s")
    wid = s * 2 + c

    mgbs = (mg0, mg1)
    scidx = (sc0, sc1)
    sss = (ss0, ss1)

    pltpu.sync_copy(zrows.at[pl.ds(s * _RPS, _RPS)],
                    acc.at[pl.ds(s * _RPS, _RPS)])

    base_e = wid * _EPW
    pltpu.sync_copy(srci.at[pl.ds(base_e, _EPW)], sidx)

    ones16 = jnp.ones((16,), jnp.float32)

    def fill_ones(e, carry):
        for j in range(8):
            mg0[e, pl.ds(j * 16, 16)] = ones16
            mg1[e, pl.ds(j * 16, 16)] = ones16
        return carry

    lax.fori_loop(0, _BB, fill_ones, 0)
    plsc.subcore_barrier()

    def drain_scatter(st):
        pltpu.make_async_copy(zrows.at[pl.ds(0, _BB)], mgbs[st],
                              sss[st]).wait()

    def scatter(b, st):
        scidx[st][...] = sidx[pl.ds(b * _BB, _BB)]
        pltpu.async_copy(mgbs[st], acc.at[scidx[st]], sss[st], add=True)

    def pair2(m, carry):
        @pl.when(m >= 1)
        def _():
            drain_scatter(0)
            drain_scatter(1)

        scatter(2 * m, 0)
        scatter(2 * m + 1, 1)
        return carry

    lax.fori_loop(0, _PAIRS, pair2, 0)
    drain_scatter(0)
    scatter(_NBLK2 - 1, 0)
    drain_scatter(0)
    drain_scatter(1)
    plsc.subcore_barrier()
    pltpu.sync_copy(acc.at[pl.ds(s * _RPS, _RPS)],
                    out_cnt.at[pl.ds(c * _NPAD + s * _RPS, _RPS)])


def _sc_cnt(src, zrows):
    mesh = plsc.VectorSubcoreMesh(core_axis_name="c", subcore_axis_name="s")
    fn = functools.partial(
        pl.kernel,
        mesh=mesh,
        out_type=jax.ShapeDtypeStruct((2 * _NPAD, _D), jnp.float32),
        scratch_types=[
            pltpu.VMEM((_EPW,), jnp.int32),
            pltpu.VMEM((_BB, _D), jnp.float32),
            pltpu.VMEM((_BB, _D), jnp.float32),
            pltpu.VMEM((_BB,), jnp.int32),
            pltpu.VMEM((_BB,), jnp.int32),
            pltpu.VMEM_SHARED((_NPAD, _D), jnp.float32),
            pltpu.SemaphoreType.DMA,
            pltpu.SemaphoreType.DMA,
        ],
    )(_sc_cnt_body)
    return fn(src, zrows)


def _node_body(p0_ref, p1_ref, c0_ref, c1_ref, x_ref, co_ref, gs_ref,
               wg_ref, wgl_ref, bg_ref, wv_ref, wvl_ref, bv_ref,
               gamma_ref, beta_ref, out_ref):
    sums = p0_ref[...] + p1_ref[...]
    cnt = c0_ref[:, :1] + c1_ref[:, :1]
    agg = jnp.where(cnt > 0, sums / jnp.maximum(cnt, 1.0), 0.0)
    cat = jnp.concatenate([x_ref[...], agg, co_ref[...]], axis=1)
    gsb = gs_ref[...]
    g = (jnp.dot(cat, wg_ref[...], preferred_element_type=jnp.float32)
         + gsb * wgl_ref[...] + bg_ref[...])
    v = (jnp.dot(cat, wv_ref[...], preferred_element_type=jnp.float32)
         + gsb * wvl_ref[...] + bv_ref[...])
    node = jax.nn.sigmoid(g) * v
    mu = jnp.mean(node, axis=-1, keepdims=True)
    var = jnp.mean((node - mu) ** 2, axis=-1, keepdims=True)
    out_ref[...] = ((node - mu) / jnp.sqrt(var + 1e-5) * gamma_ref[...]
                    + beta_ref[...])


def _node_stage(p0, p1, c0, c1, meta_xs, coords, gs2, wg_cat, wg_last, bg2,
                wv_cat, wv_last, bv2, gamma2, beta2):
    blk = 1000
    nb = _N // blk
    return pl.pallas_call(
        _node_body,
        grid=(nb,),
        in_specs=[
            pl.BlockSpec((blk, _D), lambda i: (i, 0)),
            pl.BlockSpec((blk, _D), lambda i: (i, 0)),
            pl.BlockSpec((blk, _D), lambda i: (i, 0)),
            pl.BlockSpec((blk, _D), lambda i: (i, 0)),
            pl.BlockSpec((blk, _D), lambda i: (i, 0)),
            pl.BlockSpec((blk, _D), lambda i: (i, 0)),
            pl.BlockSpec((blk, 1), lambda i: (i, 0)),
            pl.BlockSpec((3 * _D, _D), lambda i: (0, 0)),
            pl.BlockSpec((1, _D), lambda i: (0, 0)),
            pl.BlockSpec((1, _D), lambda i: (0, 0)),
            pl.BlockSpec((3 * _D, _D), lambda i: (0, 0)),
            pl.BlockSpec((1, _D), lambda i: (0, 0)),
            pl.BlockSpec((1, _D), lambda i: (0, 0)),
            pl.BlockSpec((1, _D), lambda i: (0, 0)),
            pl.BlockSpec((1, _D), lambda i: (0, 0)),
        ],
        out_specs=pl.BlockSpec((blk, _D), lambda i: (i, 0)),
        out_shape=jax.ShapeDtypeStruct((_N, _D), jnp.float32),
    )(p0, p1, c0, c1, meta_xs, coords, gs2, wg_cat, wg_last, bg2,
      wv_cat, wv_last, bv2, gamma2, beta2)


def kernel(meta_xs, edge_index, edge_feature, global_state, cells, coords,
           Wg_b, bg_b, Wv_b, bv_b, Wg_n, bg_n, Wv_n, bv_n, gamma, beta):
    del cells
    w_src = jnp.concatenate([Wg_b[:_D], Wv_b[:_D]], axis=1)
    w_dst = jnp.concatenate([Wg_b[_D:2 * _D], Wv_b[_D:2 * _D]], axis=1)

    src = edge_index[0]
    dst = edge_index[1]
    zrows = jnp.zeros((_NPAD, _D), jnp.float32)
    cnts = _sc_cnt(src, zrows)
    psrc, pdst = _project(meta_xs, w_src, w_dst)
    eb = _edge_mlp(edge_feature, Wg_b[2 * _D:], Wv_b[2 * _D:],
                   bg_b.reshape(1, _D), bv_b.reshape(1, _D))
    sums = _sc_msg(psrc, pdst, eb, src, dst, zrows)

    gs2 = global_state.reshape(_N, 1)
    p0 = lax.slice(sums, (0, 0), (_N, _D))
    p1 = lax.slice(sums, (_NPAD, 0), (_NPAD + _N, _D))
    c0 = lax.slice(cnts, (0, 0), (_N, _D))
    c1 = lax.slice(cnts, (_NPAD, 0), (_NPAD + _N, _D))
    return _node_stage(
        p0, p1, c0, c1, meta_xs, coords, gs2,
        Wg_n[:3 * _D], Wg_n[3 * _D:].reshape(1, _D), bg_n.reshape(1, _D),
        Wv_n[:3 * _D], Wv_n[3 * _D:].reshape(1, _D), bv_n.reshape(1, _D),
        gamma.reshape(1, _D), beta.reshape(1, _D),
    )


# speedup vs baseline: 1.2478x; 1.2478x over previous
import functools

import jax
import jax.numpy as jnp
from jax import lax
from jax.experimental import pallas as pl
from jax.experimental.pallas import tpu as pltpu
from jax.experimental.pallas import tpu_sc as plsc

_N = 10000
_E = 320000
_D = 128

_DW = 144
_B = 40
_NW = 32
_EPW = _E // _NW
_NBLK = _EPW // _B
_NPAD = 10240
_RPS = _NPAD // 16


def _proj_body(x_ref, ws_ref, wd_ref, ps_ref, pd_ref):
    x = x_ref[...]
    ps_ref[...] = jnp.dot(x, ws_ref[...], preferred_element_type=jnp.float32)
    pd_ref[...] = jnp.dot(x, wd_ref[...], preferred_element_type=jnp.float32)


def _project(meta_xs, w_src, w_dst):
    blk = 1000
    wspec = pl.BlockSpec((_D, 2 * _D), lambda i: (0, 0))
    ospec = pl.BlockSpec((blk, 2 * _D), lambda i: (i, 0))
    return pl.pallas_call(
        _proj_body,
        grid=(_N // blk,),
        in_specs=[pl.BlockSpec((blk, _D), lambda i: (i, 0)), wspec, wspec],
        out_specs=[ospec, ospec],
        out_shape=[
            jax.ShapeDtypeStruct((_N, 2 * _D), jnp.float32),
            jax.ShapeDtypeStruct((_N, 2 * _D), jnp.float32),
        ],
    )(meta_xs, w_src, w_dst)


def _edge_body(ef_ref, we_ref, be_ref, out_ref):
    out_ref[...] = (
        jnp.dot(ef_ref[...], we_ref[...], preferred_element_type=jnp.float32)
        + be_ref[...]
    )


def _edge_mlp(edge_feature, w_e, b_e):
    blk = 2000
    return pl.pallas_call(
        _edge_body,
        grid=(_E // blk,),
        in_specs=[
            pl.BlockSpec((blk, _D), lambda i: (i, 0)),
            pl.BlockSpec((_D, 2 * _D), lambda i: (0, 0)),
            pl.BlockSpec((1, 2 * _D), lambda i: (0, 0)),
        ],
        out_specs=pl.BlockSpec((blk, 2 * _D), lambda i: (i, 0)),
        out_shape=jax.ShapeDtypeStruct((_E, 2 * _D), jnp.float32),
    )(edge_feature, w_e, b_e)


_BB = 16
_NBLK2 = _EPW // _BB
_PAIRS = (_NBLK2 - 1) // 2


def _sc_msg_body(psrc, pdst, eb, srci, dsti, zrows, cntdep, out_sum,
                 sidx, didx, eb0, eb1, ps0, ps1, pd0, pd1, mg0, mg1,
                 sc0, sc1, acc, sg0, sg1, ss0, ss1):
    del cntdep
    c = lax.axis_index("c")
    s = lax.axis_index("s")
    wid = s * 2 + c

    ebufs = (eb0, eb1)
    psbs = (ps0, ps1)
    pdbs = (pd0, pd1)
    mgbs = (mg0, mg1)
    scidx = (sc0, sc1)
    sgs = (sg0, sg1)
    sss = (ss0, ss1)

    pltpu.sync_copy(zrows.at[pl.ds(s * _RPS, _RPS)],
                    acc.at[pl.ds(s * _RPS, _RPS)])
    plsc.subcore_barrier()

    base_e = wid * _EPW
    pltpu.sync_copy(srci.at[pl.ds(base_e, _EPW)], sidx)
    pltpu.sync_copy(dsti.at[pl.ds(base_e, _EPW)], didx)

    def issue(b, st):
        pltpu.async_copy(eb.at[pl.ds(base_e + b * _BB, _BB)], ebufs[st],
                         sgs[st])
        pltpu.async_copy(psrc.at[sidx.at[pl.ds(b * _BB, _BB)]], psbs[st],
                         sgs[st])
        pltpu.async_copy(pdst.at[didx.at[pl.ds(b * _BB, _BB)]], pdbs[st],
                         sgs[st])

    def wait_gathers(st):
        pltpu.make_async_copy(eb.at[pl.ds(0, _BB)], ebufs[st], sgs[st]).wait()
        pltpu.make_async_copy(psrc.at[pl.ds(0, _BB)], psbs[st],
                              sgs[st]).wait()
        pltpu.make_async_copy(psrc.at[pl.ds(0, _BB)], pdbs[st],
                              sgs[st]).wait()

    def drain_scatter(st):
        pltpu.make_async_copy(zrows.at[pl.ds(0, _BB)], mgbs[st],
                              sss[st]).wait()

    def compute(st):
        @plsc.parallel_loop(0, _BB, unroll=2)
        def do_edge(e):
            for j in range(8):
                o = j * 16
                g = (psbs[st][e, pl.ds(o, 16)] + pdbs[st][e, pl.ds(o, 16)]
                     + ebufs[st][e, pl.ds(o, 16)])
                v = (psbs[st][e, pl.ds(o + _D, 16)]
                     + pdbs[st][e, pl.ds(o + _D, 16)]
                     + ebufs[st][e, pl.ds(o + _D, 16)])
                mgbs[st][e, pl.ds(o, 16)] = v / (1.0 + jnp.exp(-g))

    def scatter(b, st):
        scidx[st][...] = sidx[pl.ds(b * _BB, _BB)]
        pltpu.async_copy(mgbs[st], acc.at[scidx[st]], sss[st], add=True)

    issue(0, 0)

    def pair(m, carry):
        b0 = 2 * m
        b1 = b0 + 1
        issue(b1, 1)
        wait_gathers(0)

        @pl.when(m >= 1)
        def _():
            drain_scatter(0)

        compute(0)
        scatter(b0, 0)
        issue(b0 + 2, 0)
        wait_gathers(1)

        @pl.when(m >= 1)
        def _():
            drain_scatter(1)

        compute(1)
        scatter(b1, 1)
        return carry

    lax.fori_loop(0, _PAIRS, pair, 0)
    wait_gathers(0)
    drain_scatter(0)
    compute(0)
    scatter(_NBLK2 - 1, 0)
    drain_scatter(0)
    drain_scatter(1)

    plsc.subcore_barrier()
    pltpu.sync_copy(acc.at[pl.ds(s * _RPS, _RPS)],
                    out_sum.at[pl.ds(c * _NPAD + s * _RPS, _RPS)])


def _sc_msg(psrc, pdst, eb, src, dst, zrows, cntdep):
    mesh = plsc.VectorSubcoreMesh(core_axis_name="c", subcore_axis_name="s")
    fn = functools.partial(
        pl.kernel,
        mesh=mesh,
        out_type=jax.ShapeDtypeStruct((2 * _NPAD, _D), jnp.float32),
        scratch_types=[
            pltpu.VMEM((_EPW,), jnp.int32),
            pltpu.VMEM((_EPW,), jnp.int32),
            pltpu.VMEM((_BB, 2 * _D), jnp.float32),
            pltpu.VMEM((_BB, 2 * _D), jnp.float32),
            pltpu.VMEM((_BB, 2 * _D), jnp.float32),
            pltpu.VMEM((_BB, 2 * _D), jnp.float32),
            pltpu.VMEM((_BB, 2 * _D), jnp.float32),
            pltpu.VMEM((_BB, 2 * _D), jnp.float32),
            pltpu.VMEM((_BB, _D), jnp.float32),
            pltpu.VMEM((_BB, _D), jnp.float32),
            pltpu.VMEM((_BB,), jnp.int32),
            pltpu.VMEM((_BB,), jnp.int32),
            pltpu.VMEM_SHARED((_NPAD, _D), jnp.float32),
            pltpu.SemaphoreType.DMA,
            pltpu.SemaphoreType.DMA,
            pltpu.SemaphoreType.DMA,
            pltpu.SemaphoreType.DMA,
        ],
    )(_sc_msg_body)
    return fn(psrc, pdst, eb, src, dst, zrows, cntdep)


def _sc_cnt_body(srci, zrows, out_cnt,
                 sidx, mg0, mg1, sc0, sc1, acc, ss0, ss1):
    c = lax.axis_index("c")
    s = lax.axis_index("s")
    wid = s * 2 + c

    mgbs = (mg0, mg1)
    scidx = (sc0, sc1)
    sss = (ss0, ss1)

    pltpu.sync_copy(zrows.at[pl.ds(s * _RPS, _RPS)],
                    acc.at[pl.ds(s * _RPS, _RPS)])

    base_e = wid * _EPW
    pltpu.sync_copy(srci.at[pl.ds(base_e, _EPW)], sidx)

    ones16 = jnp.ones((16,), jnp.float32)

    def fill_ones(e, carry):
        for j in range(8):
            mg0[e, pl.ds(j * 16, 16)] = ones16
            mg1[e, pl.ds(j * 16, 16)] = ones16
        return carry

    lax.fori_loop(0, _BB, fill_ones, 0)
    plsc.subcore_barrier()

    def drain_scatter(st):
        pltpu.make_async_copy(zrows.at[pl.ds(0, _BB)], mgbs[st],
                              sss[st]).wait()

    def scatter(b, st):
        scidx[st][...] = sidx[pl.ds(b * _BB, _BB)]
        pltpu.async_copy(mgbs[st], acc.at[scidx[st]], sss[st], add=True)

    def pair2(m, carry):
        @pl.when(m >= 1)
        def _():
            drain_scatter(0)
            drain_scatter(1)

        scatter(2 * m, 0)
        scatter(2 * m + 1, 1)
        return carry

    lax.fori_loop(0, _PAIRS, pair2, 0)
    drain_scatter(0)
    scatter(_NBLK2 - 1, 0)
    drain_scatter(0)
    drain_scatter(1)
    plsc.subcore_barrier()
    pltpu.sync_copy(acc.at[pl.ds(s * _RPS, _RPS)],
                    out_cnt.at[pl.ds(c * _NPAD + s * _RPS, _RPS)])


def _sc_cnt(src, zrows):
    mesh = plsc.VectorSubcoreMesh(core_axis_name="c", subcore_axis_name="s")
    fn = functools.partial(
        pl.kernel,
        mesh=mesh,
        out_type=jax.ShapeDtypeStruct((2 * _NPAD, _D), jnp.float32),
        scratch_types=[
            pltpu.VMEM((_EPW,), jnp.int32),
            pltpu.VMEM((_BB, _D), jnp.float32),
            pltpu.VMEM((_BB, _D), jnp.float32),
            pltpu.VMEM((_BB,), jnp.int32),
            pltpu.VMEM((_BB,), jnp.int32),
            pltpu.VMEM_SHARED((_NPAD, _D), jnp.float32),
            pltpu.SemaphoreType.DMA,
            pltpu.SemaphoreType.DMA,
        ],
    )(_sc_cnt_body)
    return fn(src, zrows)


def _node_body(p0_ref, p1_ref, c0_ref, c1_ref, x_ref, co_ref, gs_ref,
               wg_ref, wgl_ref, bg_ref, wv_ref, wvl_ref, bv_ref,
               gamma_ref, beta_ref, out_ref):
    sums = p0_ref[...] + p1_ref[...]
    cnt = c0_ref[:, :1] + c1_ref[:, :1]
    agg = jnp.where(cnt > 0, sums / jnp.maximum(cnt, 1.0), 0.0)
    cat = jnp.concatenate([x_ref[...], agg, co_ref[...]], axis=1)
    gsb = gs_ref[...]
    g = (jnp.dot(cat, wg_ref[...], preferred_element_type=jnp.float32)
         + gsb * wgl_ref[...] + bg_ref[...])
    v = (jnp.dot(cat, wv_ref[...], preferred_element_type=jnp.float32)
         + gsb * wvl_ref[...] + bv_ref[...])
    node = jax.nn.sigmoid(g) * v
    mu = jnp.mean(node, axis=-1, keepdims=True)
    var = jnp.mean((node - mu) ** 2, axis=-1, keepdims=True)
    out_ref[...] = ((node - mu) / jnp.sqrt(var + 1e-5) * gamma_ref[...]
                    + beta_ref[...])


def _node_stage(p0, p1, c0, c1, meta_xs, coords, gs2, wg_cat, wg_last, bg2,
                wv_cat, wv_last, bv2, gamma2, beta2):
    blk = 1000
    nb = _N // blk
    return pl.pallas_call(
        _node_body,
        grid=(nb,),
        in_specs=[
            pl.BlockSpec((blk, _D), lambda i: (i, 0)),
            pl.BlockSpec((blk, _D), lambda i: (i, 0)),
            pl.BlockSpec((blk, _D), lambda i: (i, 0)),
            pl.BlockSpec((blk, _D), lambda i: (i, 0)),
            pl.BlockSpec((blk, _D), lambda i: (i, 0)),
            pl.BlockSpec((blk, _D), lambda i: (i, 0)),
            pl.BlockSpec((blk, 1), lambda i: (i, 0)),
            pl.BlockSpec((3 * _D, _D), lambda i: (0, 0)),
            pl.BlockSpec((1, _D), lambda i: (0, 0)),
            pl.BlockSpec((1, _D), lambda i: (0, 0)),
            pl.BlockSpec((3 * _D, _D), lambda i: (0, 0)),
            pl.BlockSpec((1, _D), lambda i: (0, 0)),
            pl.BlockSpec((1, _D), lambda i: (0, 0)),
            pl.BlockSpec((1, _D), lambda i: (0, 0)),
            pl.BlockSpec((1, _D), lambda i: (0, 0)),
        ],
        out_specs=pl.BlockSpec((blk, _D), lambda i: (i, 0)),
        out_shape=jax.ShapeDtypeStruct((_N, _D), jnp.float32),
    )(p0, p1, c0, c1, meta_xs, coords, gs2, wg_cat, wg_last, bg2,
      wv_cat, wv_last, bv2, gamma2, beta2)


def kernel(meta_xs, edge_index, edge_feature, global_state, cells, coords,
           Wg_b, bg_b, Wv_b, bv_b, Wg_n, bg_n, Wv_n, bv_n, gamma, beta):
    del cells
    w_src = jnp.concatenate([Wg_b[:_D], Wv_b[:_D]], axis=1)
    w_dst = jnp.concatenate([Wg_b[_D:2 * _D], Wv_b[_D:2 * _D]], axis=1)

    w_e = jnp.concatenate([Wg_b[2 * _D:], Wv_b[2 * _D:]], axis=1)
    b_e = jnp.concatenate([bg_b, bv_b]).reshape(1, 2 * _D)
    src = edge_index[0]
    dst = edge_index[1]
    zrows = jnp.zeros((_NPAD, _D), jnp.float32)
    cnts = _sc_cnt(src, zrows)
    psrc, pdst = _project(meta_xs, w_src, w_dst)
    eb = _edge_mlp(edge_feature, w_e, b_e)
    sums = _sc_msg(psrc, pdst, eb, src, dst, zrows, cnts)

    gs2 = global_state.reshape(_N, 1)
    p0 = lax.slice(sums, (0, 0), (_N, _D))
    p1 = lax.slice(sums, (_NPAD, 0), (_NPAD + _N, _D))
    c0 = lax.slice(cnts, (0, 0), (_N, _D))
    c1 = lax.slice(cnts, (_NPAD, 0), (_NPAD + _N, _D))
    return _node_stage(
        p0, p1, c0, c1, meta_xs, coords, gs2,
        Wg_n[:3 * _D], Wg_n[3 * _D:].reshape(1, _D), bg_n.reshape(1, _D),
        Wv_n[:3 * _D], Wv_n[3 * _D:].reshape(1, _D), bv_n.reshape(1, _D),
        gamma.reshape(1, _D), beta.reshape(1, _D),
    )
